# baseline (speedup 1.0000x reference)
import functools

import jax
import jax.numpy as jnp
from jax import lax
from jax.experimental import pallas as pl
from jax.experimental.pallas import tpu as pltpu
from jax.experimental.pallas import tpu_sc as plsc

D = 64
CH = 128
K = 5
NC = 2
NS = 16
NW = NC * NS


def _emb_lookup(idx2, table, rows):
    n_chunks = idx2.shape[0]
    cpw = n_chunks // NW
    ng = cpw // K

    mesh = plsc.VectorSubcoreMesh(core_axis_name="c", subcore_axis_name="s")

    @functools.partial(
        pl.kernel,
        mesh=mesh,
        out_type=jax.ShapeDtypeStruct((rows, 2 * D), jnp.float32),
        compiler_params=pltpu.CompilerParams(use_tc_tiling_on_sc=False),
        scratch_types=[
            pltpu.VMEM((cpw, CH), jnp.int32),
            pltpu.VMEM((K, CH, D), jnp.float32),
            pltpu.VMEM((K, CH, D), jnp.float32),
            pltpu.SemaphoreType.DMA,
            pltpu.SemaphoreType.DMA,
            pltpu.SemaphoreType.DMA,
            pltpu.SemaphoreType.DMA,
        ],
    )
    def k(idx_hbm, table_hbm, out_hbm, idx_v, buf_a, buf_b, gs_a, gs_b, ss_a, ss_b):
        wid = lax.axis_index("s") * NC + lax.axis_index("c")
        cbase = wid * cpw
        pltpu.sync_copy(idx_hbm.at[pl.ds(cbase, cpw)], idx_v)

        def fire_gathers(buf, sem, g):
            for b in range(K):
                j = g * K + b
                pltpu.async_copy(table_hbm.at[idx_v.at[j]], buf.at[b], sem)

        def wait_gathers(buf, sem):
            for b in range(K):
                pltpu.make_async_copy(table_hbm.at[pl.ds(0, CH)], buf.at[b], sem).wait()

        def fire_scatters(buf, sem, g):
            for b in range(K):
                j = g * K + b
                pltpu.async_copy(
                    buf.at[b],
                    out_hbm.at[pl.ds((cbase + j) * CH, CH), pl.ds(0, D)], sem)

        def wait_scatters(buf, sem):
            for b in range(K):
                pltpu.make_async_copy(
                    buf.at[b], out_hbm.at[pl.ds(0, CH), pl.ds(0, D)], sem).wait()

        fire_gathers(buf_a, gs_a, 0)
        fire_gathers(buf_b, gs_b, 1)

        def body(t, carry):
            ga = 2 * t
            gb = ga + 1
            wait_gathers(buf_a, gs_a)
            fire_scatters(buf_a, ss_a, ga)
            wait_gathers(buf_b, gs_b)
            fire_scatters(buf_b, ss_b, gb)
            wait_scatters(buf_a, ss_a)
            fire_gathers(buf_a, gs_a, ga + 2)
            wait_scatters(buf_b, ss_b)
            fire_gathers(buf_b, gs_b, gb + 2)
            return carry

        lax.fori_loop(0, ng // 2 - 1, body, 0)

        wait_gathers(buf_a, gs_a)
        fire_scatters(buf_a, ss_a, ng - 2)
        wait_gathers(buf_b, gs_b)
        fire_scatters(buf_b, ss_b, ng - 1)
        wait_scatters(buf_a, ss_a)
        wait_scatters(buf_b, ss_b)

    return k(idx2, table)


def kernel(itemseq_input, embedding_weight):
    batch, hist = itemseq_input.shape
    rows = batch * hist
    idx2 = itemseq_input.astype(jnp.int32).reshape(rows // CH, CH)
    out = _emb_lookup(idx2, embedding_weight.astype(jnp.float32), rows)
    return out.reshape(batch, hist, 2 * D)[:, :, :D]
